# speedup vs baseline: 26.6498x; 26.6498x over previous
import jax
import jax.numpy as jnp
from jax.experimental import pallas as pl
from jax.experimental.pallas import tpu as pltpu

_H = 8
_W = 8
_EPS = 1e-6
_NB = 4096


def _aap_kernel(edges_ref, coords_ref, values_ref, Wl_ref, bl_ref, out_ref,
                acc_ref, cnt_ref):
    i = pl.program_id(0)
    nsteps = pl.num_programs(0)

    @pl.when(i == 0)
    def _init():
        acc_ref[...] = jnp.zeros_like(acc_ref)
        cnt_ref[...] = jnp.zeros_like(cnt_ref)

    x = coords_ref[:, 0]
    y = coords_ref[:, 1]
    bx = jnp.zeros(x.shape, jnp.int32)
    by = jnp.zeros(y.shape, jnp.int32)
    for j in range(1, _H):
        bx += (x >= edges_ref[0, j]).astype(jnp.int32)
        by += (y >= edges_ref[1, j]).astype(jnp.int32)
    bins = bx + _H * by

    oh = (bins[:, None] == jax.lax.broadcasted_iota(jnp.int32, (_NB, 64), 1)
          ).astype(jnp.float32)
    cnt_ref[...] += jnp.sum(oh, axis=0, keepdims=True)
    for b in range(4):
        acc_ref[b] += jax.lax.dot_general(
            oh, values_ref[b], (((0,), (0,)), ((), ())),
            preferred_element_type=jnp.float32)

    @pl.when(i == nsteps - 1)
    def _finish():
        counts = jnp.maximum(cnt_ref[0, :], 1.0)
        means = acc_ref[...] * (1.0 / counts)[None, :, None]
        flat = means.reshape(4, 64 * 64)
        out_ref[...] = flat @ Wl_ref[...] + bl_ref[...]


def kernel(values, coords, Wl, bl):
    B, N, C = values.shape
    Tx = jnp.linspace(-1.0 - _EPS, 1.0 + _EPS, _H + 1)
    Ty = jnp.linspace(-1.0 - _EPS, 1.0 + _EPS, _W + 1)
    edges = jnp.stack([Tx[:_H], Ty[:_W]]).astype(jnp.float32)
    grid = N // _NB

    return pl.pallas_call(
        _aap_kernel,
        grid=(grid,),
        in_specs=[
            pl.BlockSpec(memory_space=pltpu.SMEM),
            pl.BlockSpec((_NB, 2), lambda i: (i, 0)),
            pl.BlockSpec((B, _NB, C), lambda i: (0, i, 0)),
            pl.BlockSpec((_H * _W * C, 128), lambda i: (0, 0)),
            pl.BlockSpec((1, 128), lambda i: (0, 0)),
        ],
        out_specs=pl.BlockSpec((B, 128), lambda i: (0, 0)),
        out_shape=jax.ShapeDtypeStruct((B, 128), jnp.float32),
        scratch_shapes=[
            pltpu.VMEM((B, _H * _W, C), jnp.float32),
            pltpu.VMEM((1, _H * _W), jnp.float32),
        ],
        compiler_params=pltpu.CompilerParams(
            dimension_semantics=("arbitrary",)),
    )(edges, coords, values, Wl, bl.reshape(1, 128))


# speedup vs baseline: 27.8540x; 1.0452x over previous
import jax
import jax.numpy as jnp
from jax import lax
from jax.experimental import pallas as pl
from jax.experimental.pallas import tpu as pltpu
from jax.experimental.pallas import tpu_sc as plsc

_H = 8
_W = 8
_EPS = 1e-6
_NC = 2
_NS = 16
_NW = _NC * _NS
_L = 16
_G = 128


def _sc_body(values2d, xs, ys, edges, zacc, zcnt, ones_h,
             out_sums, out_cnt,
             xs_buf, ys_buf, vbuf, idx_buf, ones_buf, edges_buf,
             acc0, acc1, acc2, acc3, cnt_sh):
    c = lax.axis_index("c")
    s = lax.axis_index("s")
    wid = s * _NC + c
    npts = xs_buf.shape[0]
    n_total = values2d.shape[0] // 4
    base = wid * npts
    accs = (acc0, acc1, acc2, acc3)

    @pl.when(s == 0)
    def _init():
        for a in accs:
            pltpu.sync_copy(zacc, a)
        pltpu.sync_copy(zcnt, cnt_sh)

    pltpu.sync_copy(xs.at[pl.ds(base, npts)], xs_buf)
    pltpu.sync_copy(ys.at[pl.ds(base, npts)], ys_buf)
    pltpu.sync_copy(edges, edges_buf)
    pltpu.sync_copy(ones_h, ones_buf)
    plsc.subcore_barrier()

    ex = [edges_buf[j, :] for j in range(_H - 1)]
    ey = [edges_buf[_H - 1 + j, :] for j in range(_W - 1)]
    one = jnp.ones((_L,), jnp.int32)
    zero = jnp.zeros((_L,), jnp.int32)

    n_groups = npts // _G

    def group_body(g, carry):
        p0 = g * _G
        for k in range(_G // _L):
            x = xs_buf[pl.ds(p0 + k * _L, _L)]
            y = ys_buf[pl.ds(p0 + k * _L, _L)]
            bx = zero
            for e in ex:
                bx = bx + jnp.where(x >= e, one, zero)
            by = zero
            for e in ey:
                by = by + jnp.where(y >= e, one, zero)
            idx_buf[pl.ds(k * _L, _L)] = bx + _H * by
        for b in range(4):
            pltpu.sync_copy(
                values2d.at[pl.ds(b * n_total + base + p0, _G), :], vbuf)
            pltpu.sync_copy(vbuf, accs[b].at[idx_buf], add=True)
        pltpu.sync_copy(ones_buf, cnt_sh.at[idx_buf], add=True)
        return carry

    lax.fori_loop(0, n_groups, group_body, 0)

    plsc.subcore_barrier()

    @pl.when(s == 0)
    def _flush():
        nb = _H * _W
        for b in range(4):
            pltpu.sync_copy(accs[b],
                            out_sums.at[pl.ds((c * 4 + b) * nb, nb), :])
        pltpu.sync_copy(cnt_sh, out_cnt.at[pl.ds(c * nb, nb), :])


def _fin_body(sums_ref, cnt_ref, Wl_ref, bl_ref, out_ref):
    sums = sums_ref[0] + sums_ref[1]
    counts = cnt_ref[0, :, 0] + cnt_ref[1, :, 0]
    inv = 1.0 / jnp.maximum(counts, 1.0)
    means = sums * inv[None, :, None]
    out_ref[...] = means.reshape(4, _H * _W * 64) @ Wl_ref[...] + bl_ref[...]


def kernel(values, coords, Wl, bl):
    B, N, C = values.shape
    npts = N // _NW
    nb = _H * _W

    Tx = jnp.linspace(-1.0 - _EPS, 1.0 + _EPS, _H + 1)
    Ty = jnp.linspace(-1.0 - _EPS, 1.0 + _EPS, _W + 1)
    edges = jnp.repeat(
        jnp.concatenate([Tx[1:_H], Ty[1:_W]]).astype(jnp.float32)[:, None],
        _L, axis=1)
    xs = coords[:, 0] + 0.0
    ys = coords[:, 1] + 0.0
    values2d = values.reshape(B * N, C)
    zacc = jnp.zeros((nb, C), jnp.float32)
    zcnt = jnp.zeros((nb, _L), jnp.float32)
    ones_h = jnp.ones((_G, _L), jnp.float32)

    sc_call = pl.kernel(
        _sc_body,
        out_type=[
            jax.ShapeDtypeStruct((_NC * B * nb, C), jnp.float32),
            jax.ShapeDtypeStruct((_NC * nb, _L), jnp.float32),
        ],
        mesh=plsc.VectorSubcoreMesh(core_axis_name="c", subcore_axis_name="s"),
        scratch_types=[
            pltpu.VMEM((npts,), jnp.float32),
            pltpu.VMEM((npts,), jnp.float32),
            pltpu.VMEM((_G, C), jnp.float32),
            pltpu.VMEM((_G,), jnp.int32),
            pltpu.VMEM((_G, _L), jnp.float32),
            pltpu.VMEM((_H + _W - 2, _L), jnp.float32),
            pltpu.VMEM_SHARED((nb, C), jnp.float32),
            pltpu.VMEM_SHARED((nb, C), jnp.float32),
            pltpu.VMEM_SHARED((nb, C), jnp.float32),
            pltpu.VMEM_SHARED((nb, C), jnp.float32),
            pltpu.VMEM_SHARED((nb, _L), jnp.float32),
        ],
    )
    sums, cnt = sc_call(values2d, xs, ys, edges, zacc, zcnt, ones_h)

    return pl.pallas_call(
        _fin_body,
        out_shape=jax.ShapeDtypeStruct((B, 128), jnp.float32),
    )(sums.reshape(_NC, B, nb, C), cnt.reshape(_NC, nb, _L),
      Wl, bl.reshape(1, 128))


# speedup vs baseline: 36.2632x; 1.3019x over previous
import jax
import jax.numpy as jnp
from jax import lax
from jax.experimental import pallas as pl
from jax.experimental.pallas import tpu as pltpu
from jax.experimental.pallas import tpu_sc as plsc

_H = 8
_W = 8
_EPS = 1e-6
_NC = 2
_NS = 16
_NW = _NC * _NS
_L = 16
_G = 128
_B = 4


def _sc_body(values2d, xs, ys, edges, zacc, zcnt, ones_h,
             out_sums, out_cnt,
             xs_buf, ys_buf, vb_a, vb_b,
             idx_a, idx_b, ones_buf, edges_buf,
             acc0, acc1, acc2, acc3, cnt_sh,
             sem_a, sem_b):
    c = lax.axis_index("c")
    s = lax.axis_index("s")
    wid = s * _NC + c
    npts = xs_buf.shape[0]
    n_total = values2d.shape[0] // _B
    base = wid * npts
    accs = (acc0, acc1, acc2, acc3)
    vbufs = (vb_a, vb_b)
    vsems = (sem_a, sem_b)
    idxs = (idx_a, idx_b)

    @pl.when(s == 0)
    def _init():
        for a in accs:
            pltpu.sync_copy(zacc, a)
        pltpu.sync_copy(zcnt, cnt_sh)

    pltpu.sync_copy(xs.at[pl.ds(base, npts)], xs_buf)
    pltpu.sync_copy(ys.at[pl.ds(base, npts)], ys_buf)
    pltpu.sync_copy(edges, edges_buf)
    pltpu.sync_copy(ones_h, ones_buf)
    plsc.subcore_barrier()

    ex = [edges_buf[j, :] for j in range(_H - 1)]
    ey = [edges_buf[_H - 1 + j, :] for j in range(_W - 1)]
    one = jnp.ones((_L,), jnp.int32)
    zero = jnp.zeros((_L,), jnp.int32)

    n_groups = npts // _G
    n_pairs = n_groups // 2

    def compute_bins(g, bank):
        p0 = g * _G
        for k in range(_G // _L):
            x = xs_buf[pl.ds(p0 + k * _L, _L)]
            y = ys_buf[pl.ds(p0 + k * _L, _L)]
            bx = zero
            for e in ex:
                bx = bx + jnp.where(x >= e, one, zero)
            by = zero
            for e in ey:
                by = by + jnp.where(y >= e, one, zero)
            idxs[bank][pl.ds(k * _L, _L)] = bx + _H * by

    def src(g, b):
        return values2d.at[pl.ds(b * n_total + base + g * _G, _G), :]

    def fire(g, b, bank):
        pltpu.async_copy(src(g, b), vbufs[bank], vsems[bank])

    def drain(g, b, bank):
        pltpu.make_async_copy(src(g, b), vbufs[bank], vsems[bank]).wait()

    def scat(bank, b, ibank):
        pltpu.sync_copy(vbufs[bank], accs[b].at[idxs[ibank]], add=True)

    compute_bins(0, 0)
    fire(0, 0, 0)

    def pair_body(i, carry):
        g0 = 2 * i
        compute_bins(g0 + 1, 1)
        units = [(g0, b, 0) for b in range(_B)] + \
                [(g0 + 1, b, 1) for b in range(_B)]
        for u, (g, b, ibank) in enumerate(units):
            bank = u % 2
            if u + 1 < len(units):
                gn, bn, _ = units[u + 1]
                fire(gn, bn, (u + 1) % 2)
            else:
                @pl.when(i < n_pairs - 1)
                def _prefetch_next_pair():
                    compute_bins(g0 + 2, 0)
                    fire(g0 + 2, 0, 0)
            drain(g, b, bank)
            scat(bank, b, ibank)
            if b == _B - 1:
                pltpu.sync_copy(ones_buf, cnt_sh.at[idxs[ibank]], add=True)
        return carry

    lax.fori_loop(0, n_pairs, pair_body, 0)

    plsc.subcore_barrier()

    @pl.when(s == 0)
    def _flush():
        nb = _H * _W
        for b in range(_B):
            pltpu.sync_copy(accs[b],
                            out_sums.at[pl.ds((c * _B + b) * nb, nb), :])
        pltpu.sync_copy(cnt_sh, out_cnt.at[pl.ds(c * nb, nb), :])


def _fin_body(sums_ref, cnt_ref, Wl_ref, bl_ref, out_ref):
    sums = sums_ref[0] + sums_ref[1]
    counts = cnt_ref[0, :, 0] + cnt_ref[1, :, 0]
    inv = 1.0 / jnp.maximum(counts, 1.0)
    means = sums * inv[None, :, None]
    out_ref[...] = means.reshape(_B, _H * _W * 64) @ Wl_ref[...] + bl_ref[...]


def kernel(values, coords, Wl, bl):
    B, N, C = values.shape
    npts = N // _NW
    nb = _H * _W

    Tx = jnp.linspace(-1.0 - _EPS, 1.0 + _EPS, _H + 1)
    Ty = jnp.linspace(-1.0 - _EPS, 1.0 + _EPS, _W + 1)
    edges = jnp.repeat(
        jnp.concatenate([Tx[1:_H], Ty[1:_W]]).astype(jnp.float32)[:, None],
        _L, axis=1)
    xs = coords[:, 0] + 0.0
    ys = coords[:, 1] + 0.0
    values2d = values.reshape(B * N, C)
    zacc = jnp.zeros((nb, C), jnp.float32)
    zcnt = jnp.zeros((nb, _L), jnp.float32)
    ones_h = jnp.ones((_G, _L), jnp.float32)

    vbuf_t = pltpu.VMEM((_G, C), jnp.float32)
    sc_call = pl.kernel(
        _sc_body,
        out_type=[
            jax.ShapeDtypeStruct((_NC * B * nb, C), jnp.float32),
            jax.ShapeDtypeStruct((_NC * nb, _L), jnp.float32),
        ],
        mesh=plsc.VectorSubcoreMesh(core_axis_name="c", subcore_axis_name="s"),
        scratch_types=[
            pltpu.VMEM((npts,), jnp.float32),
            pltpu.VMEM((npts,), jnp.float32),
            vbuf_t, vbuf_t,
            pltpu.VMEM((_G,), jnp.int32),
            pltpu.VMEM((_G,), jnp.int32),
            pltpu.VMEM((_G, _L), jnp.float32),
            pltpu.VMEM((_H + _W - 2, _L), jnp.float32),
            pltpu.VMEM_SHARED((nb, C), jnp.float32),
            pltpu.VMEM_SHARED((nb, C), jnp.float32),
            pltpu.VMEM_SHARED((nb, C), jnp.float32),
            pltpu.VMEM_SHARED((nb, C), jnp.float32),
            pltpu.VMEM_SHARED((nb, _L), jnp.float32),
            pltpu.SemaphoreType.DMA, pltpu.SemaphoreType.DMA,
        ],
    )
    sums, cnt = sc_call(values2d, xs, ys, edges, zacc, zcnt, ones_h)

    return pl.pallas_call(
        _fin_body,
        out_shape=jax.ShapeDtypeStruct((B, 128), jnp.float32),
    )(sums.reshape(_NC, B, nb, C), cnt.reshape(_NC, nb, _L),
      Wl, bl.reshape(1, 128))
